# baseline (speedup 1.0000x reference)
import jax
import jax.numpy as jnp
from jax import lax
from jax.experimental import pallas as pl
from jax.experimental.pallas import tpu as pltpu
from jax.experimental.pallas import tpu_sc as plsc

_NC = 2
_NS = 16
_L = 16
_NW = _NC * _NS
_B = 16384
_D = 32
_BPW = _B // _NW
_CH = 128
_NCH = _BPW // _CH


def _trig_body(r_ref, c_ref, s_ref):
    c_ref[...] = jnp.cos(r_ref[...])
    s_ref[...] = jnp.sin(r_ref[...])


def _trig_tables(rel):
    cos_t, sin_t = pl.pallas_call(
        _trig_body,
        out_shape=(
            jax.ShapeDtypeStruct(rel.shape, rel.dtype),
            jax.ShapeDtypeStruct(rel.shape, rel.dtype),
        ),
    )(rel)
    return jnp.concatenate([cos_t, sin_t], axis=-1)


def _sc_body(hidx_hbm, ridx_hbm, tidx_hbm, ent_hbm, cs_hbm, out_hbm,
             hidx_v, ridx_v, tidx_v, hrows, trows, csrows, out_v, sem):
    wid = lax.axis_index("s") * _NC + lax.axis_index("c")
    row0 = wid * _NCH
    pltpu.sync_copy(hidx_hbm.at[pl.ds(row0, _NCH)], hidx_v)
    pltpu.sync_copy(ridx_hbm.at[pl.ds(row0, _NCH)], ridx_v)
    pltpu.sync_copy(tidx_hbm.at[pl.ds(row0, _NCH)], tidx_v)
    copies = []
    for k in range(_NCH):
        dst = pl.ds(k * _CH, _CH)
        copies.append(pltpu.async_copy(ent_hbm.at[hidx_v.at[k]], hrows.at[dst], sem))
        copies.append(pltpu.async_copy(ent_hbm.at[tidx_v.at[k]], trows.at[dst], sem))
        copies.append(pltpu.async_copy(cs_hbm.at[ridx_v.at[k]], csrows.at[dst], sem))
    for cp in copies:
        cp.wait()

    lane = lax.iota(jnp.int32, _L)

    def group(g, carry):
        rows = g * _L + lane
        acc = jnp.zeros((_L,), jnp.float32)
        for d in range(_D):
            col_re = jnp.full((_L,), d, jnp.int32)
            col_im = jnp.full((_L,), d + _D, jnp.int32)
            re = plsc.load_gather(hrows, [rows, col_re])
            im = plsc.load_gather(hrows, [rows, col_im])
            c = plsc.load_gather(csrows, [rows, col_re])
            s = plsc.load_gather(csrows, [rows, col_im])
            tre = plsc.load_gather(trows, [rows, col_re])
            tim = plsc.load_gather(trows, [rows, col_im])
            acc = acc + jnp.abs(re * c - im * s - tre) + jnp.abs(re * s + im * c - tim)
        plsc.store_scatter(out_v, [rows], -acc)
        return carry

    lax.fori_loop(0, _BPW // _L, group, 0)
    pltpu.sync_copy(out_v, out_hbm.at[pl.ds(wid * _BPW, _BPW)])


def _sc_call(hidx2d, ridx2d, tidx2d, ent, cs):
    mesh = plsc.VectorSubcoreMesh(
        core_axis_name="c", subcore_axis_name="s",
        num_cores=_NC, num_subcores=_NS,
    )
    return pl.kernel(
        _sc_body,
        out_type=jax.ShapeDtypeStruct((_B,), jnp.float32),
        mesh=mesh,
        compiler_params=pltpu.CompilerParams(
            needs_layout_passes=False, use_tc_tiling_on_sc=False),
        scratch_types=[
            pltpu.VMEM((_NCH, _CH), jnp.int32),
            pltpu.VMEM((_NCH, _CH), jnp.int32),
            pltpu.VMEM((_NCH, _CH), jnp.int32),
            pltpu.VMEM((_BPW, 2 * _D), jnp.float32),
            pltpu.VMEM((_BPW, 2 * _D), jnp.float32),
            pltpu.VMEM((_BPW, 2 * _D), jnp.float32),
            pltpu.VMEM((_BPW,), jnp.float32),
            pltpu.SemaphoreType.DMA,
        ],
    )(hidx2d, ridx2d, tidx2d, ent, cs)


def kernel(head_idx, relation_idx, tail_idx, entity_embeddings, relation_embeddings):
    cs = _trig_tables(relation_embeddings)
    h2 = head_idx.reshape(_NW * _NCH, _CH)
    r2 = relation_idx.reshape(_NW * _NCH, _CH)
    t2 = tail_idx.reshape(_NW * _NCH, _CH)
    return _sc_call(h2, r2, t2, entity_embeddings, cs)
